# baseline (speedup 1.0000x reference)
import functools

import jax
import jax.numpy as jnp
from jax import lax
from jax.experimental import pallas as pl
from jax.experimental.pallas import tpu as pltpu
from jax.experimental.pallas import tpu_sc as plsc

NUM_CORES = 2
NUM_SUBCORES = 16
NUM_WORKERS = NUM_CORES * NUM_SUBCORES
LANES = 16
NBUF = 3

CHUNKS = ((0, 104), (104, 96))
SEQ_PAD = 208


def _body(B, L, D, x_hbm, tab_hbm, pos_hbm, out_hbm, idx2d, idx_all, buf,
          pos_v, gsem, osem, isem):
    cid = lax.axis_index("c")
    sid = lax.axis_index("s")
    wid = sid * NUM_CORES + cid
    seq_per_w = B // NUM_WORKERS
    dreg = D // LANES
    b0 = wid * seq_per_w

    pltpu.async_copy(x_hbm.at[pl.ds(b0, seq_per_w)], idx2d, isem)
    pltpu.sync_copy(pos_hbm, pos_v)
    pltpu.make_async_copy(x_hbm.at[pl.ds(b0, seq_per_w)], idx2d, isem).wait()

    @pl.loop(0, seq_per_w)
    def _repack(s):
        for k in range(L // LANES):
            idx_all[pl.ds(s * SEQ_PAD + k * LANES, LANES)] = idx2d[
                s, pl.ds(k * LANES, LANES)
            ]
        if L % LANES:
            idx_all[pl.ds(s * SEQ_PAD + L - LANES, LANES)] = idx2d[
                s, pl.ds(L - LANES, LANES)
            ]

    def start_gather(s, slot):
        for off, n in CHUNKS:
            pltpu.async_copy(
                tab_hbm.at[idx_all.at[pl.ds(s * SEQ_PAD + off, n)]],
                buf.at[slot, pl.ds(off, n)],
                gsem.at[slot],
            )

    def wait_gather(slot):
        for off, n in CHUNKS:
            pltpu.make_async_copy(
                tab_hbm.at[idx_all.at[pl.ds(off, n)]],
                buf.at[slot, pl.ds(off, n)],
                gsem.at[slot],
            ).wait()

    def wait_out(slot):
        pltpu.make_async_copy(
            buf.at[slot], out_hbm.at[b0], osem.at[slot]
        ).wait()

    start_gather(0, 0)

    @pl.loop(0, seq_per_w)
    def _s(s):
        slot = lax.rem(s, NBUF)
        nslot = lax.rem(s + 1, NBUF)

        @pl.when(s + 1 < seq_per_w)
        def _():
            @pl.when(s >= NBUF - 1)
            def _():
                wait_out(nslot)

            start_gather(s + 1, nslot)

        wait_gather(slot)

        @pl.loop(0, L, step=4)
        def _row(i):
            for q in range(4):
                for d in range(dreg):
                    sl = pl.ds(d * LANES, LANES)
                    buf[slot, i + q, sl] = (
                        buf[slot, i + q, sl] + pos_v[i + q, sl]
                    )

        pltpu.async_copy(buf.at[slot], out_hbm.at[b0 + s], osem.at[slot])

    for r in range(NBUF):
        wait_out(r)


def kernel(x, token_table, pos_table):
    B, L = x.shape
    V, D = token_table.shape
    x = x.astype(jnp.int32)
    mesh = plsc.VectorSubcoreMesh(
        core_axis_name="c", subcore_axis_name="s", num_cores=NUM_CORES,
        num_subcores=NUM_SUBCORES,
    )
    seq_per_w = B // NUM_WORKERS
    body = functools.partial(_body, B, L, D)
    f = pl.kernel(
        body,
        out_type=jax.ShapeDtypeStruct((B, L, D), jnp.float32),
        mesh=mesh,
        scratch_types=[
            pltpu.VMEM((seq_per_w, L), jnp.int32),
            pltpu.VMEM((seq_per_w * SEQ_PAD,), jnp.int32),
            pltpu.VMEM((NBUF, L, D), jnp.float32),
            pltpu.VMEM((L, D), jnp.float32),
            pltpu.SemaphoreType.DMA((NBUF,)),
            pltpu.SemaphoreType.DMA((NBUF,)),
            pltpu.SemaphoreType.DMA,
        ],
    )
    return f(x, token_table, pos_table)


# speedup vs baseline: 2.9235x; 2.9235x over previous
import functools

import jax
import jax.numpy as jnp
from jax import lax
from jax.experimental import pallas as pl
from jax.experimental.pallas import tpu as pltpu
from jax.experimental.pallas import tpu_sc as plsc

NUM_CORES = 2
NUM_SUBCORES = 16
NUM_WORKERS = NUM_CORES * NUM_SUBCORES
LANES = 16
NBUF = 4
UNROLL = 2

CHUNK_OFF = (0, 96)
CHUNK_LEN = (96, 104)
MAXC = 104


def _body(B, L, D, x_hbm, tab_hbm, pos_hbm, out_hbm, idx_all, buf, pos_v,
          gsems, osems):
    cid = lax.axis_index("c")
    sid = lax.axis_index("s")
    wid = sid * NUM_CORES + cid
    seq_per_w = B // NUM_WORKERS
    dreg = D // LANES
    b0 = wid * seq_per_w
    n_units = 2 * seq_per_w

    pltpu.sync_copy(pos_hbm, pos_v)
    pltpu.sync_copy(x_hbm.at[pl.ds(b0 * L, seq_per_w * L)], idx_all)

    def start_gather(s, parity, slot):
        off = CHUNK_OFF[parity]
        n = CHUNK_LEN[parity]
        pltpu.async_copy(
            tab_hbm.at[idx_all.at[pl.ds(s * L + off, n)]],
            buf.at[slot, pl.ds(0, n)],
            gsems[slot],
        )

    def wait_gather(parity, slot):
        n = CHUNK_LEN[parity]
        pltpu.make_async_copy(
            tab_hbm.at[idx_all.at[pl.ds(0, n)]],
            buf.at[slot, pl.ds(0, n)],
            gsems[slot],
        ).wait()

    def wait_out(parity, slot):
        off = CHUNK_OFF[parity]
        n = CHUNK_LEN[parity]
        pltpu.make_async_copy(
            buf.at[slot, pl.ds(0, n)],
            out_hbm.at[b0, pl.ds(off, n)],
            osems[slot],
        ).wait()

    def step(s, r):
        parity = r % 2
        off = CHUNK_OFF[parity]
        n = CHUNK_LEN[parity]
        nslot = (r + 2) % NBUF

        u = 2 * s + parity

        @pl.when(u + 2 < n_units)
        def _():
            @pl.when(u >= 2)
            def _():
                wait_out(parity, nslot)

            start_gather(s + 1, parity, nslot)

        wait_gather(parity, r)

        @pl.loop(0, n, step=UNROLL)
        def _row(i):
            for q in range(UNROLL):
                for d in range(dreg):
                    sl = pl.ds(d * LANES, LANES)
                    buf[r, i + q, sl] = buf[r, i + q, sl] + pos_v[off + i + q, sl]

        pltpu.async_copy(
            buf.at[r, pl.ds(0, n)],
            out_hbm.at[b0 + s, pl.ds(off, n)],
            osems[r],
        )

    start_gather(0, 0, 0)
    start_gather(0, 1, 1)

    @pl.loop(0, seq_per_w, step=2)
    def _s(s0):
        for r in range(NBUF):
            step(s0 + r // 2, r)

    for r in range(NBUF):
        wait_out(r % 2, r)


def kernel(x, token_table, pos_table):
    B, L = x.shape
    V, D = token_table.shape
    x = x.astype(jnp.int32).reshape(B * L)
    mesh = plsc.VectorSubcoreMesh(
        core_axis_name="c", subcore_axis_name="s", num_cores=NUM_CORES,
        num_subcores=NUM_SUBCORES,
    )
    seq_per_w = B // NUM_WORKERS
    body = functools.partial(_body, B, L, D)
    f = pl.kernel(
        body,
        out_type=jax.ShapeDtypeStruct((B, L, D), jnp.float32),
        mesh=mesh,
        scratch_types=[
            pltpu.VMEM((seq_per_w * L,), jnp.int32),
            pltpu.VMEM((NBUF, MAXC, D), jnp.float32),
            pltpu.VMEM((L, D), jnp.float32),
            [pltpu.SemaphoreType.DMA] * NBUF,
            [pltpu.SemaphoreType.DMA] * NBUF,
        ],
    )
    return f(x, token_table, pos_table)


# speedup vs baseline: 2.9929x; 1.0237x over previous
import functools

import jax
import jax.numpy as jnp
from jax import lax
from jax.experimental import pallas as pl
from jax.experimental.pallas import tpu as pltpu
from jax.experimental.pallas import tpu_sc as plsc

NUM_CORES = 2
NUM_SUBCORES = 16
NUM_WORKERS = NUM_CORES * NUM_SUBCORES
LANES = 16
NBUF = 4
UNROLL = 2

CHUNK_OFF = (0, 96)
CHUNK_LEN = (96, 104)
MAXC = 104


def _body(B, L, D, x_hbm, tab_hbm, pos_hbm, out_hbm, idx_all, buf, pos_v,
          gsems, osems, isem, psem):
    cid = lax.axis_index("c")
    sid = lax.axis_index("s")
    wid = sid * NUM_CORES + cid
    seq_per_w = B // NUM_WORKERS
    dreg = D // LANES
    b0 = wid * seq_per_w
    n_units = 2 * seq_per_w

    pltpu.async_copy(x_hbm.at[pl.ds(b0 * L, seq_per_w * L)], idx_all, isem)
    pltpu.async_copy(pos_hbm, pos_v, psem)

    def start_gather(s, parity, slot):
        off = CHUNK_OFF[parity]
        n = CHUNK_LEN[parity]
        pltpu.async_copy(
            tab_hbm.at[idx_all.at[pl.ds(s * L + off, n)]],
            buf.at[slot, pl.ds(0, n)],
            gsems[slot],
        )

    def wait_gather(parity, slot):
        n = CHUNK_LEN[parity]
        pltpu.make_async_copy(
            tab_hbm.at[idx_all.at[pl.ds(0, n)]],
            buf.at[slot, pl.ds(0, n)],
            gsems[slot],
        ).wait()

    def wait_out(parity, slot):
        off = CHUNK_OFF[parity]
        n = CHUNK_LEN[parity]
        pltpu.make_async_copy(
            buf.at[slot, pl.ds(0, n)],
            out_hbm.at[b0, pl.ds(off, n)],
            osems[slot],
        ).wait()

    def step(s, r):
        parity = r % 2
        off = CHUNK_OFF[parity]
        n = CHUNK_LEN[parity]
        nslot = (r + 2) % NBUF

        u = 2 * s + parity

        @pl.when(u + 2 < n_units)
        def _():
            @pl.when(u >= 2)
            def _():
                wait_out(parity, nslot)

            start_gather(s + 1, parity, nslot)

        wait_gather(parity, r)

        @pl.loop(0, n, step=UNROLL)
        def _row(i):
            for q in range(UNROLL):
                for d in range(dreg):
                    sl = pl.ds(d * LANES, LANES)
                    buf[r, i + q, sl] = buf[r, i + q, sl] + pos_v[off + i + q, sl]

        pltpu.async_copy(
            buf.at[r, pl.ds(0, n)],
            out_hbm.at[b0 + s, pl.ds(off, n)],
            osems[r],
        )

    pltpu.make_async_copy(
        x_hbm.at[pl.ds(b0 * L, seq_per_w * L)], idx_all, isem
    ).wait()
    start_gather(0, 0, 0)
    start_gather(0, 1, 1)
    pltpu.make_async_copy(pos_hbm, pos_v, psem).wait()

    @pl.loop(0, seq_per_w, step=2)
    def _s(s0):
        for r in range(NBUF):
            step(s0 + r // 2, r)

    for r in range(NBUF):
        wait_out(r % 2, r)


def kernel(x, token_table, pos_table):
    B, L = x.shape
    V, D = token_table.shape
    x = x.astype(jnp.int32).reshape(B * L)
    mesh = plsc.VectorSubcoreMesh(
        core_axis_name="c", subcore_axis_name="s", num_cores=NUM_CORES,
        num_subcores=NUM_SUBCORES,
    )
    seq_per_w = B // NUM_WORKERS
    body = functools.partial(_body, B, L, D)
    f = pl.kernel(
        body,
        out_type=jax.ShapeDtypeStruct((B, L, D), jnp.float32),
        mesh=mesh,
        scratch_types=[
            pltpu.VMEM((seq_per_w * L,), jnp.int32),
            pltpu.VMEM((NBUF, MAXC, D), jnp.float32),
            pltpu.VMEM((L, D), jnp.float32),
            [pltpu.SemaphoreType.DMA] * NBUF,
            [pltpu.SemaphoreType.DMA] * NBUF,
            pltpu.SemaphoreType.DMA,
            pltpu.SemaphoreType.DMA,
        ],
    )
    return f(x, token_table, pos_table)
